# Initial kernel scaffold; baseline (speedup 1.0000x reference)
import jax
import jax.numpy as jnp
from jax.experimental import pallas as pl


def kernel(inputs, w):
    raise NotImplementedError("write your pallas kernel here")


# speedup vs baseline: 1.1089x; 1.1089x over previous
import jax
import jax.numpy as jnp
from jax.experimental import pallas as pl
from jax.experimental.pallas import tpu as pltpu

EMBED_DIM = 64
NUM_CODES = 8192
TOKENS = 8192
BLOCK_ROWS = 128
NUM_BLOCKS = TOKENS // BLOCK_ROWS
COMMIT = 0.25


def _vq_body(x_ref, x2_ref, w_ref, wt_ref,
             enc_ref, idx_ref, qskip_ref, loss_ref, perp_ref,
             counts_ref, lsum_ref):
    i = pl.program_id(0)

    @pl.when(i == 0)
    def _init():
        lsum_ref[0] = 0.0
        counts_ref[:] = jnp.zeros_like(counts_ref)

    x = x_ref[:]
    w = w_ref[:]
    b = jnp.dot(x, w, preferred_element_type=jnp.float32)
    w2 = jnp.sum(w * w, axis=0, keepdims=True)
    neg = -((x2_ref[:] - 2.0 * b) + w2)

    iota = jax.lax.broadcasted_iota(jnp.int32, (BLOCK_ROWS, NUM_CODES), 1)
    half = NUM_CODES // 2
    n0 = neg[:, :half]
    n1 = neg[:, half:]
    m0 = jnp.max(n0, axis=1, keepdims=True)
    m1 = jnp.max(n1, axis=1, keepdims=True)
    i0 = jnp.min(jnp.where(n0 == m0, iota[:, :half], NUM_CODES), axis=1)
    i1 = jnp.min(jnp.where(n1 == m1, iota[:, half:], NUM_CODES), axis=1)
    m0r = m0.astype(jnp.bfloat16).astype(jnp.float32)
    take1 = (m1 >= m0r)[:, 0]
    idx = jnp.where(take1, i1, i0)

    onehot = (iota == idx[:, None]).astype(jnp.float32)
    enc_ref[:] = onehot
    idx_ref[0, 0, :] = idx

    q = jnp.dot(onehot, wt_ref[:], preferred_element_type=jnp.float32)
    qskip_ref[:] = x + (q - x)

    diff = q - x
    lsum_ref[0] += jnp.sum(diff * diff)
    counts_ref[:] += jnp.sum(onehot, axis=0, keepdims=True)

    @pl.when(i == NUM_BLOCKS - 1)
    def _fin():
        mean_sq = lsum_ref[0] / (TOKENS * EMBED_DIM)
        loss_ref[0, 0] = mean_sq + COMMIT * mean_sq
        avg = counts_ref[:] / TOKENS
        ent = jnp.sum(avg * jnp.log(avg + 1e-10))
        perp_ref[0, 0] = jnp.exp(-ent)


def kernel(inputs, w):
    flat = inputs.reshape(-1, EMBED_DIM)
    wt = w.T
    x2 = jnp.sum(flat ** 2, axis=1, keepdims=True)

    enc, idx3, qskip, loss, perp = pl.pallas_call(
        _vq_body,
        grid=(NUM_BLOCKS,),
        in_specs=[
            pl.BlockSpec((BLOCK_ROWS, EMBED_DIM), lambda i: (i, 0)),
            pl.BlockSpec((BLOCK_ROWS, 1), lambda i: (i, 0)),
            pl.BlockSpec((EMBED_DIM, NUM_CODES), lambda i: (0, 0)),
            pl.BlockSpec((NUM_CODES, EMBED_DIM), lambda i: (0, 0)),
        ],
        out_specs=[
            pl.BlockSpec((BLOCK_ROWS, NUM_CODES), lambda i: (i, 0)),
            pl.BlockSpec((1, 1, BLOCK_ROWS), lambda i: (i, 0, 0)),
            pl.BlockSpec((BLOCK_ROWS, EMBED_DIM), lambda i: (i, 0)),
            pl.BlockSpec(memory_space=pltpu.SMEM,
                         block_shape=(1, 1), index_map=lambda i: (0, 0)),
            pl.BlockSpec(memory_space=pltpu.SMEM,
                         block_shape=(1, 1), index_map=lambda i: (0, 0)),
        ],
        out_shape=[
            jax.ShapeDtypeStruct((TOKENS, NUM_CODES), jnp.float32),
            jax.ShapeDtypeStruct((NUM_BLOCKS, 1, BLOCK_ROWS), jnp.int32),
            jax.ShapeDtypeStruct((TOKENS, EMBED_DIM), jnp.float32),
            jax.ShapeDtypeStruct((1, 1), jnp.float32),
            jax.ShapeDtypeStruct((1, 1), jnp.float32),
        ],
        scratch_shapes=[
            pltpu.VMEM((1, NUM_CODES), jnp.float32),
            pltpu.SMEM((1,), jnp.float32),
        ],
        compiler_params=pltpu.CompilerParams(
            dimension_semantics=("arbitrary",),
        ),
    )(flat, x2, w, wt)

    quantized_skip = qskip.reshape(inputs.shape)
    encoding_indices = idx3.reshape(inputs.shape[:-1])
    return (quantized_skip, loss[0, 0], perp[0, 0], enc, encoding_indices)
